# baseline (device time: 36710 ns/iter reference)
import jax
import jax.numpy as jnp
from jax import lax
from jax.experimental import pallas as pl
from jax.experimental.pallas import tpu as pltpu


def kernel(partial, resid, gamma):
    m, d = resid.shape
    half = m // 2

    def body(part_ref, resid_ref, gamma_ref, out_ref,
             mine, other, sem_sa, sem_ra, sem_sb, sem_rb):
        my_x = lax.axis_index("x")
        my_y = lax.axis_index("y")
        x_nbr = (1 - my_x, my_y)
        y_nbr = (my_x, 1 - my_y)

        barrier = pltpu.get_barrier_semaphore()
        for nbr in (x_nbr, y_nbr):
            pl.semaphore_signal(barrier, inc=1, device_id=nbr,
                                device_id_type=pl.DeviceIdType.MESH)
        pl.semaphore_wait(barrier, 2)

        mine[...] = part_ref[0].astype(jnp.bfloat16)

        row0 = my_y * half
        rdma_a = pltpu.make_async_remote_copy(
            src_ref=mine.at[pl.ds(row0, half)],
            dst_ref=other.at[pl.ds(row0, half)],
            send_sem=sem_sa, recv_sem=sem_ra,
            device_id=x_nbr, device_id_type=pl.DeviceIdType.MESH,
        )
        rdma_a.start()
        rdma_a.wait()

        rdma_b = pltpu.make_async_remote_copy(
            src_ref=other.at[pl.ds(row0, half)],
            dst_ref=other.at[pl.ds(row0, half)],
            send_sem=sem_sb, recv_sem=sem_rb,
            device_id=y_nbr, device_id_type=pl.DeviceIdType.MESH,
        )
        rdma_b.start()
        rdma_b.wait()

        y = part_ref[0] + other[...].astype(jnp.float32) + resid_ref[...]
        ms = jnp.mean(y * y, axis=-1, keepdims=True)
        out_ref[...] = y * lax.rsqrt(ms + 1e-6) * gamma_ref[...]

    return pl.pallas_call(
        body,
        out_shape=jax.ShapeDtypeStruct((m, d), jnp.float32),
        in_specs=[pl.BlockSpec(memory_space=pltpu.VMEM)] * 3,
        out_specs=pl.BlockSpec(memory_space=pltpu.VMEM),
        scratch_shapes=[
            pltpu.VMEM((m, d), jnp.bfloat16),
            pltpu.VMEM((m, d), jnp.bfloat16),
            pltpu.SemaphoreType.DMA,
            pltpu.SemaphoreType.DMA,
            pltpu.SemaphoreType.DMA,
            pltpu.SemaphoreType.DMA,
        ],
        compiler_params=pltpu.CompilerParams(collective_id=0),
    )(partial, resid, gamma.reshape(1, d))


# device time: 26201 ns/iter; 1.4011x vs baseline; 1.4011x over previous
import jax
import jax.numpy as jnp
from jax import lax
from jax.experimental import pallas as pl
from jax.experimental.pallas import tpu as pltpu

C = 8


def kernel(partial, resid, gamma):
    m, d = resid.shape
    half = m // 2
    rows = half // C

    def body(part_ref, resid_ref, gamma_ref, out_ref,
             mine, other, sa, ra, sb, rb):
        my_x = lax.axis_index("x")
        my_y = lax.axis_index("y")
        x_nbr = (1 - my_x, my_y)
        y_nbr = (my_x, 1 - my_y)

        barrier = pltpu.get_barrier_semaphore()
        for nbr in (x_nbr, y_nbr):
            pl.semaphore_signal(barrier, inc=1, device_id=nbr,
                                device_id_type=pl.DeviceIdType.MESH)
        pl.semaphore_wait(barrier, 2)

        row0 = my_y * half
        orow0 = (1 - my_y) * half

        a_rdmas = []
        for c in range(C):
            mine[pl.ds(c * rows, rows), :] = (
                part_ref[0, pl.ds(row0 + c * rows, rows), :]
                .astype(jnp.bfloat16)
            )
            r = pltpu.make_async_remote_copy(
                src_ref=mine.at[pl.ds(c * rows, rows)],
                dst_ref=other.at[pl.ds(row0 + c * rows, rows)],
                send_sem=sa.at[c], recv_sem=ra.at[c],
                device_id=x_nbr, device_id_type=pl.DeviceIdType.MESH,
            )
            r.start()
            a_rdmas.append(r)

        def norm_rows(r0):
            yv = (part_ref[0, pl.ds(r0, rows), :]
                  + other[pl.ds(r0, rows), :].astype(jnp.float32)
                  + resid_ref[pl.ds(r0, rows), :])
            ms = jnp.mean(yv * yv, axis=-1, keepdims=True)
            out_ref[pl.ds(r0, rows), :] = (
                yv * lax.rsqrt(ms + 1e-6) * gamma_ref[...]
            )

        b_rdmas = []
        for c in range(C):
            a_rdmas[c].wait_recv()
            r = pltpu.make_async_remote_copy(
                src_ref=other.at[pl.ds(row0 + c * rows, rows)],
                dst_ref=other.at[pl.ds(row0 + c * rows, rows)],
                send_sem=sb.at[c], recv_sem=rb.at[c],
                device_id=y_nbr, device_id_type=pl.DeviceIdType.MESH,
            )
            r.start()
            b_rdmas.append(r)
            norm_rows(row0 + c * rows)

        for c in range(C):
            b_rdmas[c].wait_recv()
            norm_rows(orow0 + c * rows)

        for c in range(C):
            a_rdmas[c].wait_send()
            b_rdmas[c].wait_send()

    return pl.pallas_call(
        body,
        out_shape=jax.ShapeDtypeStruct((m, d), jnp.float32),
        in_specs=[pl.BlockSpec(memory_space=pltpu.VMEM)] * 3,
        out_specs=pl.BlockSpec(memory_space=pltpu.VMEM),
        scratch_shapes=[
            pltpu.VMEM((half, d), jnp.bfloat16),
            pltpu.VMEM((m, d), jnp.bfloat16),
            pltpu.SemaphoreType.DMA((C,)),
            pltpu.SemaphoreType.DMA((C,)),
            pltpu.SemaphoreType.DMA((C,)),
            pltpu.SemaphoreType.DMA((C,)),
        ],
        compiler_params=pltpu.CompilerParams(collective_id=0),
    )(partial, resid, gamma.reshape(1, d))


# device time: 26137 ns/iter; 1.4045x vs baseline; 1.0024x over previous
import jax
import jax.numpy as jnp
from jax import lax
from jax.experimental import pallas as pl
from jax.experimental.pallas import tpu as pltpu

C = 8


def kernel(partial, resid, gamma):
    m, d = resid.shape
    half = m // 2
    rows = half // C

    def body(part_ref, resid_ref, gamma_ref, out_ref,
             mine, other_half, out_bf, recv_out, sa, ra, sb, rb):
        my_x = lax.axis_index("x")
        my_y = lax.axis_index("y")
        x_nbr = (1 - my_x, my_y)
        y_nbr = (my_x, 1 - my_y)

        barrier = pltpu.get_barrier_semaphore()
        for nbr in (x_nbr, y_nbr):
            pl.semaphore_signal(barrier, inc=1, device_id=nbr,
                                device_id_type=pl.DeviceIdType.MESH)
        pl.semaphore_wait(barrier, 2)

        row0 = my_y * half
        orow0 = (1 - my_y) * half

        a_rdmas = []
        for c in range(C):
            lo = c * rows
            mine[pl.ds(lo, rows), :] = (
                part_ref[0, pl.ds(row0 + lo, rows), :].astype(jnp.bfloat16)
            )
            r = pltpu.make_async_remote_copy(
                src_ref=mine.at[pl.ds(lo, rows)],
                dst_ref=other_half.at[pl.ds(lo, rows)],
                send_sem=sa.at[c], recv_sem=ra.at[c],
                device_id=x_nbr, device_id_type=pl.DeviceIdType.MESH,
            )
            r.start()
            a_rdmas.append(r)

        b_rdmas = []
        for c in range(C):
            lo = c * rows
            a_rdmas[c].wait_recv()
            yv = (part_ref[0, pl.ds(row0 + lo, rows), :]
                  + other_half[pl.ds(lo, rows), :].astype(jnp.float32)
                  + resid_ref[pl.ds(row0 + lo, rows), :])
            ms = jnp.mean(yv * yv, axis=-1, keepdims=True)
            scaled = yv * lax.rsqrt(ms + 1e-6) * gamma_ref[...]
            out_ref[pl.ds(row0 + lo, rows), :] = scaled
            out_bf[pl.ds(lo, rows), :] = scaled.astype(jnp.bfloat16)
            r = pltpu.make_async_remote_copy(
                src_ref=out_bf.at[pl.ds(lo, rows)],
                dst_ref=recv_out.at[pl.ds(lo, rows)],
                send_sem=sb.at[c], recv_sem=rb.at[c],
                device_id=y_nbr, device_id_type=pl.DeviceIdType.MESH,
            )
            r.start()
            b_rdmas.append(r)

        for c in range(C):
            lo = c * rows
            b_rdmas[c].wait_recv()
            out_ref[pl.ds(orow0 + lo, rows), :] = (
                recv_out[pl.ds(lo, rows), :].astype(jnp.float32)
            )

        for c in range(C):
            a_rdmas[c].wait_send()
            b_rdmas[c].wait_send()

    return pl.pallas_call(
        body,
        out_shape=jax.ShapeDtypeStruct((m, d), jnp.float32),
        in_specs=[pl.BlockSpec(memory_space=pltpu.VMEM)] * 3,
        out_specs=pl.BlockSpec(memory_space=pltpu.VMEM),
        scratch_shapes=[
            pltpu.VMEM((half, d), jnp.bfloat16),
            pltpu.VMEM((half, d), jnp.bfloat16),
            pltpu.VMEM((half, d), jnp.bfloat16),
            pltpu.VMEM((half, d), jnp.bfloat16),
            pltpu.SemaphoreType.DMA((C,)),
            pltpu.SemaphoreType.DMA((C,)),
            pltpu.SemaphoreType.DMA((C,)),
            pltpu.SemaphoreType.DMA((C,)),
        ],
        compiler_params=pltpu.CompilerParams(collective_id=0),
    )(partial, resid, gamma.reshape(1, d))
